# baseline (speedup 1.0000x reference)
import numpy as np
import jax
import jax.numpy as jnp
from jax.experimental import pallas as pl
from jax.experimental.pallas import tpu as pltpu

_H = 32
_W = 32
_K = 5
_CU = 32
_CN = 3
_G = 1


def _elu(x):
    return jnp.where(x > 0, x, jnp.exp(jnp.minimum(x, 0.0)) - 1.0)


def _shift(x, s, L):
    if s == 0:
        return x
    return pltpu.roll(x, shift=(-s) % L, axis=1)


def _make_stage1(L):
    K, P = _K, _K // 2
    SX, SY = _G, _W * _G

    def body(u_ref, masks_ref, w1_ref, b1_ref, wz_ref, b2_ref, wl_ref, bl_ref,
             x3_ref, tsum_ref):
        x0 = 2.0 * u_ref[...] - 1.0

        cols = []
        for kx in range(K):
            dx = kx - P
            s = _shift(x0, dx * SX, L)
            if dx != 0:
                s = s * masks_ref[kx:kx + 1, :]
            cols.append(s)
        p1x = jnp.concatenate(cols, axis=0)
        blocks = []
        for ky in range(K):
            dy = ky - P
            b = _shift(p1x, dy * SY, L)
            if dy != 0:
                b = b * masks_ref[K + ky:K + ky + 1, :]
            blocks.append(b)
        p1 = jnp.concatenate(blocks, axis=0)
        p1h = p1.astype(jnp.bfloat16)
        p1l = (p1 - p1h.astype(jnp.float32)).astype(jnp.bfloat16)
        p75 = jnp.concatenate([p1h, p1l, p1h], axis=0)
        x1 = _elu(jnp.dot(w1_ref[...], p75,
                          preferred_element_type=jnp.float32) + b1_ref[...])

        cols = []
        for kx in range(K):
            dx = kx - P
            s = _shift(x1, dx * SX, L)
            if dx != 0:
                s = s * masks_ref[kx:kx + 1, :]
            cols.append(s.astype(jnp.bfloat16))
        p2 = jnp.concatenate(cols, axis=0)
        z = jnp.dot(wz_ref[...], p2,
                    preferred_element_type=jnp.float32)
        acc = z[2 * _CU:3 * _CU, :]
        for ky in range(K):
            dy = ky - P
            if dy == 0:
                continue
            t = _shift(z[ky * _CU:(ky + 1) * _CU, :], dy * SY, L)
            acc = acc + t * masks_ref[K + ky:K + ky + 1, :]
        x2 = _elu(acc + b2_ref[...])

        x3 = _elu(jnp.dot(wl_ref[...], x2,
                          preferred_element_type=jnp.float32) + bl_ref[...])
        x3_ref[...] = x3
        tsum_ref[...] = jnp.sum(x3, axis=1, keepdims=True)[None]

    return body


def _make_stage2(inv_count):
    def body(x_ref, tsum_ref, o_ref):
        mean = jnp.sum(tsum_ref[...], axis=0) * inv_count
        d = x_ref[...] - mean
        o_ref[...] = jnp.where(d > 0, 1.0, jnp.where(d < 0, -1.0, 0.0))
    return body


def kernel(u_message, real_cpu, conv_w_0, conv_b_0, conv_w_1, conv_b_1, lin_w, lin_b):
    del real_cpu
    N, Ck, H, W = u_message.shape
    assert (Ck, H, W) == (1, _H, _W)
    HW = H * W
    K, Cu, Cn, p = _K, _CU, _CN, _K // 2

    bt = 16
    while N % bt:
        bt //= 2
    L = bt * HW
    nt = N // bt

    g = _G
    u2 = jnp.transpose(u_message.reshape(N // g, g, HW), (0, 2, 1))
    u2 = u2.reshape(1, N * HW).astype(jnp.float32)

    w1p = jnp.transpose(conv_w_0, (2, 3, 0, 1)).reshape(K * K, Cu).T
    w1h = w1p.astype(jnp.bfloat16)
    w1l = (w1p - w1h.astype(jnp.float32)).astype(jnp.bfloat16)
    w1p = jnp.concatenate([w1h, w1h, w1l], axis=1)
    wz = jnp.transpose(conv_w_1, (2, 0, 3, 1)).reshape(K * Cu, K * Cu)
    wz = wz.astype(jnp.bfloat16)
    wl = lin_w.reshape(Cn, Cu)
    b1 = conv_b_0.reshape(Cu, 1)
    b2 = conv_b_1.reshape(Cu, 1)
    bl = lin_b.reshape(Cn, 1)

    xs = (np.arange(HW * g) // g) % W
    ys = (np.arange(HW * g) // (W * g)) % H
    m = np.ones((2 * K, HW * g), np.float32)
    for kx in range(K):
        dx = kx - p
        m[kx] = ((xs + dx >= 0) & (xs + dx < W)).astype(np.float32)
    for ky in range(K):
        dy = ky - p
        m[K + ky] = ((ys + dy >= 0) & (ys + dy < H)).astype(np.float32)
    masks = jnp.asarray(np.tile(m, (1, bt // g)))

    const_spec = lambda a: pl.BlockSpec(a.shape, lambda i, nd=a.ndim: (0,) * nd)

    x3, tsum = pl.pallas_call(
        _make_stage1(L),
        out_shape=(jax.ShapeDtypeStruct((Cn, N * HW), jnp.float32),
                   jax.ShapeDtypeStruct((nt, Cn, 1), jnp.float32)),
        grid=(nt,),
        in_specs=[pl.BlockSpec((1, L), lambda i: (0, i)),
                  const_spec(masks), const_spec(w1p), const_spec(b1),
                  const_spec(wz), const_spec(b2), const_spec(wl),
                  const_spec(bl)],
        out_specs=(pl.BlockSpec((Cn, L), lambda i: (0, i)),
                   pl.BlockSpec((1, Cn, 1), lambda i: (i, 0, 0))),
        compiler_params=pltpu.CompilerParams(
            dimension_semantics=("parallel",)),
    )(u2, masks, w1p, b1, wz, b2, wl, bl)

    L2 = N * HW
    nt2 = 1
    while L2 > 65536:
        L2 //= 2
        nt2 *= 2
    out_flat = pl.pallas_call(
        _make_stage2(1.0 / (N * HW)),
        out_shape=jax.ShapeDtypeStruct((Cn, N * HW), jnp.float32),
        grid=(nt2,),
        in_specs=[pl.BlockSpec((Cn, L2), lambda i: (0, i)),
                  const_spec(tsum)],
        out_specs=pl.BlockSpec((Cn, L2), lambda i: (0, i)),
        compiler_params=pltpu.CompilerParams(
            dimension_semantics=("parallel",)),
    )(x3, tsum)

    out = out_flat.reshape(Cn, N // g, HW, g)
    return jnp.transpose(out, (1, 3, 0, 2)).reshape(N, Cn, H, W)


# speedup vs baseline: 1.0490x; 1.0490x over previous
import numpy as np
import jax
import jax.numpy as jnp
from jax.experimental import pallas as pl
from jax.experimental.pallas import tpu as pltpu

_H = 32
_W = 32
_K = 5
_CU = 32
_CN = 3
_G = 1


def _elu(x):
    return jnp.where(x > 0, x, jnp.exp(jnp.minimum(x, 0.0)) - 1.0)


def _shift(x, s, L):
    if s == 0:
        return x
    return pltpu.roll(x, shift=(-s) % L, axis=1)


def _make_stage1(L):
    K, P = _K, _K // 2
    SX, SY = _G, _W * _G

    def body(u_ref, masks_ref, w1_ref, b1_ref, wz_ref, b2_ref, wl_ref, bl_ref,
             x3_ref, tsum_ref):
        x0 = 2.0 * u_ref[...] - 1.0

        cols = []
        for kx in range(K):
            dx = kx - P
            s = _shift(x0, dx * SX, L)
            if dx != 0:
                s = s * masks_ref[kx:kx + 1, :]
            cols.append(s)
        p1x = jnp.concatenate(cols, axis=0)
        blocks = []
        for ky in range(K):
            dy = ky - P
            b = _shift(p1x, dy * SY, L)
            if dy != 0:
                b = b * masks_ref[K + ky:K + ky + 1, :]
            blocks.append(b)
        p1 = jnp.concatenate(blocks, axis=0)
        x1 = _elu(jnp.dot(w1_ref[...], p1,
                          preferred_element_type=jnp.float32,
                          precision=jax.lax.Precision.HIGHEST) + b1_ref[...])

        cols = []
        for kx in range(K):
            dx = kx - P
            s = _shift(x1, dx * SX, L)
            if dx != 0:
                s = s * masks_ref[kx:kx + 1, :]
            cols.append(s.astype(jnp.bfloat16))
        p2 = jnp.concatenate(cols, axis=0)
        z = jnp.dot(wz_ref[...], p2,
                    preferred_element_type=jnp.float32)
        acc = z[2 * _CU:3 * _CU, :]
        for ky in range(K):
            dy = ky - P
            if dy == 0:
                continue
            t = _shift(z[ky * _CU:(ky + 1) * _CU, :], dy * SY, L)
            acc = acc + t * masks_ref[K + ky:K + ky + 1, :]
        x2 = _elu(acc + b2_ref[...])

        x3 = _elu(jnp.dot(wl_ref[...], x2,
                          preferred_element_type=jnp.float32) + bl_ref[...])
        x3_ref[...] = x3
        tsum_ref[...] = jnp.sum(x3, axis=1, keepdims=True)[None]

    return body


def _make_stage2(inv_count):
    def body(x_ref, tsum_ref, o_ref):
        mean = jnp.sum(tsum_ref[...], axis=0) * inv_count
        d = x_ref[...] - mean
        o_ref[...] = jnp.where(d > 0, 1.0, jnp.where(d < 0, -1.0, 0.0))
    return body


def kernel(u_message, real_cpu, conv_w_0, conv_b_0, conv_w_1, conv_b_1, lin_w, lin_b):
    del real_cpu
    N, Ck, H, W = u_message.shape
    assert (Ck, H, W) == (1, _H, _W)
    HW = H * W
    K, Cu, Cn, p = _K, _CU, _CN, _K // 2

    bt = 32
    while N % bt:
        bt //= 2
    L = bt * HW
    nt = N // bt

    g = _G
    u2 = jnp.transpose(u_message.reshape(N // g, g, HW), (0, 2, 1))
    u2 = u2.reshape(1, N * HW).astype(jnp.float32)

    w1p = jnp.transpose(conv_w_0, (2, 3, 0, 1)).reshape(K * K, Cu).T
    wz = jnp.transpose(conv_w_1, (2, 0, 3, 1)).reshape(K * Cu, K * Cu)
    wz = wz.astype(jnp.bfloat16)
    wl = lin_w.reshape(Cn, Cu)
    b1 = conv_b_0.reshape(Cu, 1)
    b2 = conv_b_1.reshape(Cu, 1)
    bl = lin_b.reshape(Cn, 1)

    xs = (np.arange(HW * g) // g) % W
    ys = (np.arange(HW * g) // (W * g)) % H
    m = np.ones((2 * K, HW * g), np.float32)
    for kx in range(K):
        dx = kx - p
        m[kx] = ((xs + dx >= 0) & (xs + dx < W)).astype(np.float32)
    for ky in range(K):
        dy = ky - p
        m[K + ky] = ((ys + dy >= 0) & (ys + dy < H)).astype(np.float32)
    masks = jnp.asarray(np.tile(m, (1, bt // g)))

    const_spec = lambda a: pl.BlockSpec(a.shape, lambda i, nd=a.ndim: (0,) * nd)

    x3, tsum = pl.pallas_call(
        _make_stage1(L),
        out_shape=(jax.ShapeDtypeStruct((Cn, N * HW), jnp.float32),
                   jax.ShapeDtypeStruct((nt, Cn, 1), jnp.float32)),
        grid=(nt,),
        in_specs=[pl.BlockSpec((1, L), lambda i: (0, i)),
                  const_spec(masks), const_spec(w1p), const_spec(b1),
                  const_spec(wz), const_spec(b2), const_spec(wl),
                  const_spec(bl)],
        out_specs=(pl.BlockSpec((Cn, L), lambda i: (0, i)),
                   pl.BlockSpec((1, Cn, 1), lambda i: (i, 0, 0))),
        compiler_params=pltpu.CompilerParams(
            dimension_semantics=("parallel",)),
    )(u2, masks, w1p, b1, wz, b2, wl, bl)

    L2 = N * HW
    nt2 = 1
    while L2 > 65536:
        L2 //= 2
        nt2 *= 2
    out_flat = pl.pallas_call(
        _make_stage2(1.0 / (N * HW)),
        out_shape=jax.ShapeDtypeStruct((Cn, N * HW), jnp.float32),
        grid=(nt2,),
        in_specs=[pl.BlockSpec((Cn, L2), lambda i: (0, i)),
                  const_spec(tsum)],
        out_specs=pl.BlockSpec((Cn, L2), lambda i: (0, i)),
        compiler_params=pltpu.CompilerParams(
            dimension_semantics=("parallel",)),
    )(x3, tsum)

    out = out_flat.reshape(Cn, N // g, HW, g)
    return jnp.transpose(out, (1, 3, 0, 2)).reshape(N, Cn, H, W)


# speedup vs baseline: 1.2373x; 1.1795x over previous
import numpy as np
import jax
import jax.numpy as jnp
from jax.experimental import pallas as pl
from jax.experimental.pallas import tpu as pltpu

_H = 32
_W = 32
_K = 5
_CU = 32
_CN = 3
_G = 1


def _elu(x):
    return jnp.where(x > 0, x, jnp.exp(jnp.minimum(x, 0.0)) - 1.0)


def _shift(x, s, L):
    if s == 0:
        return x
    return pltpu.roll(x, shift=(-s) % L, axis=1)


def _make_stage1(L):
    K, P = _K, _K // 2
    SX, SY = _G, _W * _G

    def body(u_ref, masks_ref, maski_ref, w1_ref, b1_ref, wz_ref, b2_ref,
             wl_ref, bl_ref, x3_ref, tsum_ref):
        x0 = 2.0 * u_ref[...] - 1.0

        cols = []
        for kx in range(K):
            dx = kx - P
            s = _shift(x0, dx * SX, L)
            if dx != 0:
                s = s * masks_ref[kx:kx + 1, :]
            cols.append(s)
        p1x = jnp.concatenate(cols, axis=0)
        blocks = []
        for ky in range(K):
            dy = ky - P
            b = _shift(p1x, dy * SY, L)
            if dy != 0:
                b = b * masks_ref[K + ky:K + ky + 1, :]
            blocks.append(b)
        p1 = jnp.concatenate(blocks, axis=0)
        x1 = _elu(jnp.dot(w1_ref[...], p1,
                          preferred_element_type=jnp.float32,
                          precision=jax.lax.Precision.HIGHEST) + b1_ref[...])

        x1i = pltpu.bitcast(x1.astype(jnp.bfloat16), jnp.int32)
        cols = []
        for kx in range(K):
            dx = kx - P
            if dx == 0:
                cols.append(x1i)
                continue
            si = pltpu.roll(x1i, shift=(-dx * SX) % L, axis=1)
            j = kx if kx < K // 2 else kx - 1
            cols.append(jnp.bitwise_and(si, maski_ref[j:j + 1, :]))
        p2 = pltpu.bitcast(jnp.concatenate(cols, axis=0),
                           jnp.bfloat16)
        z = jnp.dot(wz_ref[...], p2,
                    preferred_element_type=jnp.float32)
        acc = z[2 * _CU:3 * _CU, :]
        for ky in range(K):
            dy = ky - P
            if dy == 0:
                continue
            t = _shift(z[ky * _CU:(ky + 1) * _CU, :], dy * SY, L)
            acc = acc + t * masks_ref[K + ky:K + ky + 1, :]
        x2 = _elu(acc + b2_ref[...])

        x3 = _elu(jnp.dot(wl_ref[...], x2,
                          preferred_element_type=jnp.float32) + bl_ref[...])
        x3_ref[...] = x3
        tsum_ref[...] = jnp.sum(x3, axis=1, keepdims=True)[None]

    return body


def _make_stage2(inv_count):
    def body(x_ref, tsum_ref, o_ref):
        mean = jnp.sum(tsum_ref[...], axis=0) * inv_count
        d = x_ref[...] - mean
        o_ref[...] = jnp.where(d > 0, 1.0, jnp.where(d < 0, -1.0, 0.0))
    return body


def kernel(u_message, real_cpu, conv_w_0, conv_b_0, conv_w_1, conv_b_1, lin_w, lin_b):
    del real_cpu
    N, Ck, H, W = u_message.shape
    assert (Ck, H, W) == (1, _H, _W)
    HW = H * W
    K, Cu, Cn, p = _K, _CU, _CN, _K // 2

    bt = 32
    while N % bt:
        bt //= 2
    L = bt * HW
    nt = N // bt

    g = _G
    u2 = jnp.transpose(u_message.reshape(N // g, g, HW), (0, 2, 1))
    u2 = u2.reshape(1, N * HW).astype(jnp.float32)

    w1p = jnp.transpose(conv_w_0, (2, 3, 0, 1)).reshape(K * K, Cu).T
    wz = jnp.transpose(conv_w_1, (2, 0, 3, 1)).reshape(K * Cu, K * Cu)
    wz = wz.astype(jnp.bfloat16)
    wl = lin_w.reshape(Cn, Cu)
    b1 = conv_b_0.reshape(Cu, 1)
    b2 = conv_b_1.reshape(Cu, 1)
    bl = lin_b.reshape(Cn, 1)

    xs = (np.arange(HW * g) // g) % W
    ys = (np.arange(HW * g) // (W * g)) % H
    m = np.ones((2 * K, HW * g), np.float32)
    for kx in range(K):
        dx = kx - p
        m[kx] = ((xs + dx >= 0) & (xs + dx < W)).astype(np.float32)
    for ky in range(K):
        dy = ky - p
        m[K + ky] = ((ys + dy >= 0) & (ys + dy < H)).astype(np.float32)
    masks = jnp.asarray(np.tile(m, (1, bt // g)))
    mi = m[[0, 1, 3, 4], :].astype(bool)
    maski = jnp.asarray(np.tile(np.where(mi, -1, 0).astype(np.int32),
                                (1, bt // g)))

    const_spec = lambda a: pl.BlockSpec(a.shape, lambda i, nd=a.ndim: (0,) * nd)

    x3, tsum = pl.pallas_call(
        _make_stage1(L),
        out_shape=(jax.ShapeDtypeStruct((Cn, N * HW), jnp.float32),
                   jax.ShapeDtypeStruct((nt, Cn, 1), jnp.float32)),
        grid=(nt,),
        in_specs=[pl.BlockSpec((1, L), lambda i: (0, i)),
                  const_spec(masks), const_spec(maski), const_spec(w1p),
                  const_spec(b1), const_spec(wz), const_spec(b2),
                  const_spec(wl), const_spec(bl)],
        out_specs=(pl.BlockSpec((Cn, L), lambda i: (0, i)),
                   pl.BlockSpec((1, Cn, 1), lambda i: (i, 0, 0))),
        compiler_params=pltpu.CompilerParams(
            dimension_semantics=("parallel",)),
    )(u2, masks, maski, w1p, b1, wz, b2, wl, bl)

    L2 = N * HW
    nt2 = 1
    while L2 > 65536:
        L2 //= 2
        nt2 *= 2
    out_flat = pl.pallas_call(
        _make_stage2(1.0 / (N * HW)),
        out_shape=jax.ShapeDtypeStruct((Cn, N * HW), jnp.float32),
        grid=(nt2,),
        in_specs=[pl.BlockSpec((Cn, L2), lambda i: (0, i)),
                  const_spec(tsum)],
        out_specs=pl.BlockSpec((Cn, L2), lambda i: (0, i)),
        compiler_params=pltpu.CompilerParams(
            dimension_semantics=("parallel",)),
    )(x3, tsum)

    out = out_flat.reshape(Cn, N // g, HW, g)
    return jnp.transpose(out, (1, 3, 0, 2)).reshape(N, Cn, H, W)


# speedup vs baseline: 1.2440x; 1.0055x over previous
import numpy as np
import jax
import jax.numpy as jnp
from jax.experimental import pallas as pl
from jax.experimental.pallas import tpu as pltpu

_H = 32
_W = 32
_K = 5
_CU = 32
_CN = 3
_G = 1


def _elu(x):
    return jnp.where(x > 0, x, jnp.exp(jnp.minimum(x, 0.0)) - 1.0)


def _shift(x, s, L):
    if s == 0:
        return x
    return pltpu.roll(x, shift=(-s) % L, axis=1)


def _make_stage1(L):
    K, P = _K, _K // 2
    SX, SY = _G, _W * _G

    def body(u_ref, masks_ref, maski_ref, w1_ref, b1_ref, wz_ref, b2_ref,
             wl_ref, bl_ref, x3_ref, tsum_ref):
        x0 = 2.0 * u_ref[...] - 1.0

        cols = []
        for kx in range(K):
            dx = kx - P
            s = _shift(x0, dx * SX, L)
            if dx != 0:
                s = s * masks_ref[kx:kx + 1, :]
            cols.append(s)
        p1x = jnp.concatenate(cols, axis=0)
        blocks = []
        for ky in range(K):
            dy = ky - P
            b = _shift(p1x, dy * SY, L)
            if dy != 0:
                b = b * masks_ref[K + ky:K + ky + 1, :]
            blocks.append(b)
        p1 = jnp.concatenate(blocks, axis=0)
        x1 = _elu(jnp.dot(w1_ref[...], p1,
                          preferred_element_type=jnp.float32,
                          precision=jax.lax.Precision.HIGHEST) + b1_ref[...])

        x1i = pltpu.bitcast(x1.astype(jnp.bfloat16), jnp.int32)
        cols = []
        for kx in range(K):
            dx = kx - P
            if dx == 0:
                cols.append(x1i)
                continue
            si = pltpu.roll(x1i, shift=(-dx * SX) % L, axis=1)
            j = kx if kx < K // 2 else kx - 1
            cols.append(jnp.bitwise_and(si, maski_ref[j:j + 1, :]))
        p2 = pltpu.bitcast(jnp.concatenate(cols, axis=0),
                           jnp.bfloat16)
        z = jnp.dot(wz_ref[...], p2,
                    preferred_element_type=jnp.float32)
        acc = z[2 * _CU:3 * _CU, :]
        for ky in range(K):
            dy = ky - P
            if dy == 0:
                continue
            t = _shift(z[ky * _CU:(ky + 1) * _CU, :], dy * SY, L)
            acc = acc + t * masks_ref[K + ky:K + ky + 1, :]
        x2 = _elu(acc + b2_ref[...])

        x3 = _elu(jnp.dot(wl_ref[...], x2,
                          preferred_element_type=jnp.float32) + bl_ref[...])
        x3_ref[...] = x3
        tsum_ref[...] = jnp.sum(x3, axis=1, keepdims=True)[None]

    return body


def _make_stage2(inv_count):
    def body(x_ref, tsum_ref, o_ref):
        mean = jnp.sum(tsum_ref[...], axis=0) * inv_count
        d = x_ref[...] - mean
        o_ref[...] = jnp.where(d > 0, 1.0, jnp.where(d < 0, -1.0, 0.0))
    return body


def kernel(u_message, real_cpu, conv_w_0, conv_b_0, conv_w_1, conv_b_1, lin_w, lin_b):
    del real_cpu
    N, Ck, H, W = u_message.shape
    assert (Ck, H, W) == (1, _H, _W)
    HW = H * W
    K, Cu, Cn, p = _K, _CU, _CN, _K // 2

    bt = 32
    while N % bt:
        bt //= 2
    L = bt * HW
    nt = N // bt

    g = _G
    u2 = jnp.transpose(u_message.reshape(N // g, g, HW), (0, 2, 1))
    u2 = u2.reshape(1, N * HW).astype(jnp.float32)

    w1p = jnp.transpose(conv_w_0, (2, 3, 0, 1)).reshape(K * K, Cu).T
    wz = jnp.transpose(conv_w_1, (2, 0, 3, 1)).reshape(K * Cu, K * Cu)
    wz = wz.astype(jnp.bfloat16)
    wl = lin_w.reshape(Cn, Cu)
    b1 = conv_b_0.reshape(Cu, 1)
    b2 = conv_b_1.reshape(Cu, 1)
    bl = lin_b.reshape(Cn, 1)

    xs = (np.arange(HW * g) // g) % W
    ys = (np.arange(HW * g) // (W * g)) % H
    m = np.ones((2 * K, HW * g), np.float32)
    for kx in range(K):
        dx = kx - p
        m[kx] = ((xs + dx >= 0) & (xs + dx < W)).astype(np.float32)
    for ky in range(K):
        dy = ky - p
        m[K + ky] = ((ys + dy >= 0) & (ys + dy < H)).astype(np.float32)
    masks = jnp.asarray(np.tile(m, (1, bt // g)))
    mi = m[[0, 1, 3, 4], :].astype(bool)
    maski = jnp.asarray(np.tile(np.where(mi, -1, 0).astype(np.int32),
                                (1, bt // g)))

    const_spec = lambda a: pl.BlockSpec(a.shape, lambda i, nd=a.ndim: (0,) * nd)

    x3, tsum = pl.pallas_call(
        _make_stage1(L),
        out_shape=(jax.ShapeDtypeStruct((Cn, N * HW), jnp.float32),
                   jax.ShapeDtypeStruct((nt, Cn, 1), jnp.float32)),
        grid=(nt,),
        in_specs=[pl.BlockSpec((1, L), lambda i: (0, i)),
                  const_spec(masks), const_spec(maski), const_spec(w1p),
                  const_spec(b1), const_spec(wz), const_spec(b2),
                  const_spec(wl), const_spec(bl)],
        out_specs=(pl.BlockSpec((Cn, L), lambda i: (0, i)),
                   pl.BlockSpec((1, Cn, 1), lambda i: (i, 0, 0))),
        compiler_params=pltpu.CompilerParams(
            dimension_semantics=("parallel",)),
    )(u2, masks, maski, w1p, b1, wz, b2, wl, bl)

    L2 = N * HW
    nt2 = 1
    while L2 > 262144:
        L2 //= 2
        nt2 *= 2
    out_flat = pl.pallas_call(
        _make_stage2(1.0 / (N * HW)),
        out_shape=jax.ShapeDtypeStruct((Cn, N * HW), jnp.float32),
        grid=(nt2,),
        in_specs=[pl.BlockSpec((Cn, L2), lambda i: (0, i)),
                  const_spec(tsum)],
        out_specs=pl.BlockSpec((Cn, L2), lambda i: (0, i)),
        compiler_params=pltpu.CompilerParams(
            dimension_semantics=("parallel",)),
    )(x3, tsum)

    out = out_flat.reshape(Cn, N // g, HW, g)
    return jnp.transpose(out, (1, 3, 0, 2)).reshape(N, Cn, H, W)
